# initial kernel scaffold (unmeasured)
import jax
import jax.numpy as jnp
from jax import lax
from jax.experimental import pallas as pl
from jax.experimental.pallas import tpu as pltpu


def kernel(A, B):
    A = A.astype(jnp.bfloat16)
    B = B.astype(jnp.bfloat16)
    M, K = A.shape
    N = B.shape[1]

    def body(a_ref, b_ref, out_ref, recv_ref, send_sem, recv_sem):
        my_x = lax.axis_index("x")
        my_y = lax.axis_index("y")
        peer = (my_x, 1 - my_y)

        out_ref[...] = jnp.dot(
            a_ref[...], b_ref[...], preferred_element_type=jnp.float32
        ).astype(jnp.bfloat16)

        rdma = pltpu.make_async_remote_copy(
            src_ref=out_ref,
            dst_ref=recv_ref,
            send_sem=send_sem,
            recv_sem=recv_sem,
            device_id=peer,
            device_id_type=pl.DeviceIdType.MESH,
        )
        rdma.start()
        rdma.wait()

        out_ref[...] = (
            out_ref[...].astype(jnp.float32) + recv_ref[...].astype(jnp.float32)
        ).astype(jnp.bfloat16)

    return pl.pallas_call(
        body,
        out_shape=jax.ShapeDtypeStruct((M, N), jnp.bfloat16),
        in_specs=[
            pl.BlockSpec(memory_space=pltpu.VMEM),
            pl.BlockSpec(memory_space=pltpu.VMEM),
        ],
        out_specs=pl.BlockSpec(memory_space=pltpu.VMEM),
        scratch_shapes=[
            pltpu.VMEM((M, N), jnp.bfloat16),
            pltpu.SemaphoreType.DMA,
            pltpu.SemaphoreType.DMA,
        ],
    )(A, B)


# baseline (device time: 290353 ns/iter reference)
import jax
import jax.numpy as jnp
from jax import lax
from jax.experimental import pallas as pl
from jax.experimental.pallas import tpu as pltpu


def kernel(A, B):
    A = A.astype(jnp.bfloat16)
    B = B.astype(jnp.bfloat16)
    M, K = A.shape
    N = B.shape[1]

    def body(a_ref, b_ref, out_ref, recv_ref, send_sem, recv_sem):
        my_x = lax.axis_index("x")
        my_y = lax.axis_index("y")
        peer = (my_x, 1 - my_y)

        out_ref[...] = jnp.dot(
            a_ref[...], b_ref[...], preferred_element_type=jnp.float32
        ).astype(jnp.bfloat16)

        rdma = pltpu.make_async_remote_copy(
            src_ref=out_ref,
            dst_ref=recv_ref,
            send_sem=send_sem,
            recv_sem=recv_sem,
            device_id=peer,
            device_id_type=pl.DeviceIdType.MESH,
        )
        rdma.start()
        rdma.wait()

        out_ref[...] = (
            out_ref[...].astype(jnp.float32) + recv_ref[...].astype(jnp.float32)
        ).astype(jnp.bfloat16)

    return pl.pallas_call(
        body,
        out_shape=jax.ShapeDtypeStruct((M, N), jnp.bfloat16),
        in_specs=[
            pl.BlockSpec(memory_space=pltpu.VMEM),
            pl.BlockSpec(memory_space=pltpu.VMEM),
        ],
        out_specs=pl.BlockSpec(memory_space=pltpu.VMEM),
        scratch_shapes=[
            pltpu.VMEM((M, N), jnp.bfloat16),
            pltpu.SemaphoreType.DMA,
            pltpu.SemaphoreType.DMA,
        ],
        compiler_params=pltpu.CompilerParams(
            vmem_limit_bytes=120 * 1024 * 1024,
        ),
    )(A, B)


# device time: 257021 ns/iter; 1.1297x vs baseline; 1.1297x over previous
import jax
import jax.numpy as jnp
from jax import lax
from jax.experimental import pallas as pl
from jax.experimental.pallas import tpu as pltpu

NT = 8


def kernel(A, B):
    A = A.astype(jnp.bfloat16)
    B = B.astype(jnp.bfloat16)
    M, K = A.shape
    N = B.shape[1]
    TN = N // NT

    def body(a_ref, b_ref, out_ref, recv_ref, send_sems, recv_sems):
        my_x = lax.axis_index("x")
        my_y = lax.axis_index("y")
        peer = (my_x, 1 - my_y)

        barrier = pltpu.get_barrier_semaphore()
        pl.semaphore_signal(
            barrier, inc=1, device_id=peer, device_id_type=pl.DeviceIdType.MESH
        )
        pl.semaphore_wait(barrier, 1)

        rdmas = []
        for j in range(NT):
            sl = pl.ds(j * TN, TN)
            out_ref[:, sl] = jnp.dot(
                a_ref[...], b_ref[:, sl], preferred_element_type=jnp.float32
            ).astype(jnp.bfloat16)
            rdma = pltpu.make_async_remote_copy(
                src_ref=out_ref.at[:, sl],
                dst_ref=recv_ref.at[:, sl],
                send_sem=send_sems.at[j],
                recv_sem=recv_sems.at[j],
                device_id=peer,
                device_id_type=pl.DeviceIdType.MESH,
            )
            rdma.start()
            rdmas.append(rdma)

        for j in range(NT):
            sl = pl.ds(j * TN, TN)
            rdmas[j].wait_send()
            rdmas[j].wait_recv()
            out_ref[:, sl] = (
                out_ref[:, sl].astype(jnp.float32)
                + recv_ref[:, sl].astype(jnp.float32)
            ).astype(jnp.bfloat16)

    return pl.pallas_call(
        body,
        out_shape=jax.ShapeDtypeStruct((M, N), jnp.bfloat16),
        in_specs=[
            pl.BlockSpec(memory_space=pltpu.VMEM),
            pl.BlockSpec(memory_space=pltpu.VMEM),
        ],
        out_specs=pl.BlockSpec(memory_space=pltpu.VMEM),
        scratch_shapes=[
            pltpu.VMEM((M, N), jnp.bfloat16),
            pltpu.SemaphoreType.DMA((NT,)),
            pltpu.SemaphoreType.DMA((NT,)),
        ],
        compiler_params=pltpu.CompilerParams(
            vmem_limit_bytes=120 * 1024 * 1024,
            collective_id=0,
        ),
    )(A, B)


# device time: 231818 ns/iter; 1.2525x vs baseline; 1.1087x over previous
import jax
import jax.numpy as jnp
from jax import lax
from jax.experimental import pallas as pl
from jax.experimental.pallas import tpu as pltpu

MB_ = 4
NB = 6


def kernel(A, B):
    M, K = A.shape
    N = B.shape[1]
    RB = M // MB_
    CB = N // NB
    NTILES = MB_ * NB

    def body(
        a_ref, b_ref, out_ref, recv_hbm,
        a16_ref, a_stage, b_stage, b16_ref, recv_stage,
        a_sems, b_sems, r_sems, send_sems, recv_sems,
    ):
        my_x = lax.axis_index("x")
        my_y = lax.axis_index("y")
        peer = (my_x, 1 - my_y)

        def a_copy(i, slot):
            return pltpu.make_async_copy(
                a_ref.at[pl.ds(i * RB, RB), :], a_stage.at[slot], a_sems.at[slot]
            )

        def b_copy(j, slot):
            return pltpu.make_async_copy(
                b_ref.at[:, pl.ds(j * CB, CB)], b_stage.at[slot], b_sems.at[slot]
            )

        a_copy(0, 0).start()
        b_copy(0, 0).start()

        barrier = pltpu.get_barrier_semaphore()
        pl.semaphore_signal(
            barrier, inc=1, device_id=peer, device_id_type=pl.DeviceIdType.MESH
        )
        pl.semaphore_wait(barrier, 1)

        rdmas = {}
        for j in range(NB):
            cs = pl.ds(j * CB, CB)
            b_copy(j, j % 2).wait()
            if j + 1 < NB:
                b_copy(j + 1, (j + 1) % 2).start()
            b16_ref[j % 2] = b_stage[j % 2].astype(jnp.bfloat16)
            for i in range(MB_):
                rs = pl.ds(i * RB, RB)
                if j == 0:
                    a_copy(i, i % 2).wait()
                    if i + 1 < MB_:
                        a_copy(i + 1, (i + 1) % 2).start()
                    a16_ref[rs, :] = a_stage[i % 2].astype(jnp.bfloat16)
                out_ref[rs, cs] = jnp.dot(
                    a16_ref[rs, :], b16_ref[j % 2],
                    preferred_element_type=jnp.float32,
                ).astype(jnp.bfloat16)
                t = j * MB_ + i
                rdma = pltpu.make_async_remote_copy(
                    src_ref=out_ref.at[rs, cs],
                    dst_ref=recv_hbm.at[rs, cs],
                    send_sem=send_sems.at[t],
                    recv_sem=recv_sems.at[t],
                    device_id=peer,
                    device_id_type=pl.DeviceIdType.MESH,
                )
                rdma.start()
                rdmas[t] = rdma

        for j in range(NB):
            cs = pl.ds(j * CB, CB)
            for i in range(MB_):
                rs = pl.ds(i * RB, RB)
                t = j * MB_ + i
                rdmas[t].wait_recv()
                cp = pltpu.make_async_copy(
                    recv_hbm.at[rs, cs], recv_stage.at[t % 2], r_sems.at[t % 2]
                )
                cp.start()
                cp.wait()
                rdmas[t].wait_send()
                out_ref[rs, cs] = (
                    out_ref[rs, cs].astype(jnp.float32)
                    + recv_stage[t % 2].astype(jnp.float32)
                ).astype(jnp.bfloat16)

    out, _ = pl.pallas_call(
        body,
        out_shape=(
            jax.ShapeDtypeStruct((M, N), jnp.bfloat16),
            jax.ShapeDtypeStruct((M, N), jnp.bfloat16),
        ),
        in_specs=[
            pl.BlockSpec(memory_space=pltpu.MemorySpace.HBM),
            pl.BlockSpec(memory_space=pltpu.MemorySpace.HBM),
        ],
        out_specs=(
            pl.BlockSpec(memory_space=pltpu.MemorySpace.VMEM),
            pl.BlockSpec(memory_space=pltpu.MemorySpace.HBM),
        ),
        scratch_shapes=[
            pltpu.VMEM((M, K), jnp.bfloat16),
            pltpu.VMEM((2, RB, K), jnp.float32),
            pltpu.VMEM((2, K, CB), jnp.float32),
            pltpu.VMEM((2, K, CB), jnp.bfloat16),
            pltpu.VMEM((2, RB, CB), jnp.bfloat16),
            pltpu.SemaphoreType.DMA((2,)),
            pltpu.SemaphoreType.DMA((2,)),
            pltpu.SemaphoreType.DMA((2,)),
            pltpu.SemaphoreType.DMA((NTILES,)),
            pltpu.SemaphoreType.DMA((NTILES,)),
        ],
        compiler_params=pltpu.CompilerParams(
            vmem_limit_bytes=60 * 1024 * 1024,
            collective_id=0,
        ),
    )(A, B)
    return out
